# baseline (speedup 1.0000x reference)
import jax
import jax.numpy as jnp
from jax.experimental import pallas as pl
from jax.experimental.pallas import tpu as pltpu

IN_F = 768
OUT_F = 768
N_BANKS = 64
N_ROWS = 128
NBUF = 10


def _body(sel_smem, selv_ref, x_ref, bias_ref, w_hbm, out_ref,
          acc, xt_s, uniq, seen, wbuf, sems):
    def copy_bank(bank, slot):
        return pltpu.make_async_copy(
            w_hbm.at[bank], wbuf.at[slot], sems.at[slot])

    def copy_in(i, slot):
        return copy_bank(uniq[i], slot)

    copy_bank(sel_smem[0], 0).start()

    def init_seen(b, c):
        seen[b] = 0
        uniq[b] = N_BANKS
        return c
    jax.lax.fori_loop(0, N_BANKS, init_seen, 0)
    seen[sel_smem[0]] = 1
    uniq[0] = sel_smem[0]

    def scan_p(p, cnt):
        b = sel_smem[p]
        new = seen[b] == 0

        @pl.when(new)
        def _():
            seen[b] = 1
            uniq[cnt] = b

        return cnt + jnp.where(new, 1, 0)

    nd = jax.lax.fori_loop(1, N_ROWS, scan_p, 1)

    for i in range(1, NBUF):
        @pl.when(i < nd)
        def _(i=i):
            copy_in(i, i).start()

    sel = selv_ref[...]

    onehot = (
        jax.lax.broadcasted_iota(jnp.int32, (N_BANKS, N_ROWS), 0) == sel
    ).astype(jnp.float32)
    acc[...] = jax.lax.dot_general(
        bias_ref[...], onehot, (((0,), (0,)), ((), ())),
        preferred_element_type=jnp.float32)

    xt_s[...] = x_ref[...].T

    def matpart(i, slot):
        y = jax.lax.dot_general(
            wbuf[slot], xt_s[...], (((1,), (0,)), ((), ())),
            preferred_element_type=jnp.float32)
        return jnp.where(sel == uniq[i], y, 0.0)

    def refill(i, slot):
        @pl.when(i + NBUF < nd)
        def _():
            copy_in(i + NBUF, slot).start()

    def step(half, carry):
        i0 = half * 2
        i1 = i0 + 1
        s0 = jax.lax.rem(i0, NBUF)
        s1 = jax.lax.rem(i1, NBUF)
        copy_in(i0, s0).wait()

        @pl.when(i1 < nd)
        def _():
            copy_in(i1, s1).wait()

        acc[...] += matpart(i0, s0) + matpart(i1, s1)
        refill(i0, s0)
        refill(i1, s1)
        return carry

    jax.lax.fori_loop(0, (nd + 1) // 2, step, 0)

    out_ref[...] = acc[...].T


def kernel(tensor, bank_selections, weight, bias):
    x = tensor.reshape(N_ROWS, IN_F)
    flat = bank_selections.reshape(N_ROWS).astype(jnp.int32)
    selv = flat.reshape(1, N_ROWS)

    out = pl.pallas_call(
        _body,
        in_specs=[
            pl.BlockSpec(memory_space=pltpu.SMEM),
            pl.BlockSpec(memory_space=pltpu.VMEM),
            pl.BlockSpec(memory_space=pltpu.VMEM),
            pl.BlockSpec(memory_space=pltpu.VMEM),
            pl.BlockSpec(memory_space=pl.ANY),
        ],
        out_specs=pl.BlockSpec(memory_space=pltpu.VMEM),
        out_shape=jax.ShapeDtypeStruct((N_ROWS, OUT_F), jnp.float32),
        scratch_shapes=[
            pltpu.VMEM((OUT_F, N_ROWS), jnp.float32),
            pltpu.VMEM((IN_F, N_ROWS), jnp.float32),
            pltpu.SMEM((N_BANKS,), jnp.int32),
            pltpu.SMEM((N_BANKS,), jnp.int32),
            pltpu.VMEM((NBUF, OUT_F, IN_F), jnp.float32),
            pltpu.SemaphoreType.DMA((NBUF,)),
        ],
    )(flat, selv, x, bias, weight)

    return out.reshape(tensor.shape[0], tensor.shape[1], OUT_F)
